# baseline (speedup 1.0000x reference)
import functools

import jax
import jax.numpy as jnp
from jax import lax
from jax.experimental import pallas as pl
from jax.experimental.pallas import tpu as pltpu
from jax.experimental.pallas import tpu_sc as plsc

_NUM_CORES = 2
_NUM_SUBCORES = 16
_NUM_WORKERS = _NUM_CORES * _NUM_SUBCORES

_CHUNK = 64


def _fuse_table_kernel(table_ref, wt_ref, bias_ref, m_ref):
    m_ref[...] = (
        jnp.dot(table_ref[...], wt_ref[...], preferred_element_type=jnp.float32)
        + bias_ref[...]
    )


def _gather_body(m_hbm, ids_hbm, out_hbm, idx_v, rows_v, sem):
    wid = lax.axis_index("s") * _NUM_CORES + lax.axis_index("c")
    n = ids_hbm.shape[0]
    n_per_w = n // _NUM_WORKERS
    base = wid * n_per_w

    def chunk_step(c, carry):
        off = pl.multiple_of(base + c * _CHUNK, 8)
        pltpu.sync_copy(ids_hbm.at[pl.ds(off, _CHUNK)], idx_v)
        pltpu.async_copy(m_hbm.at[idx_v], rows_v, sem).wait()
        pltpu.sync_copy(rows_v, out_hbm.at[pl.ds(off, _CHUNK)])
        return carry

    lax.fori_loop(0, n_per_w // _CHUNK, chunk_step, 0)


def kernel(input_ids, embed_table, proj_w, proj_b):
    b, l = input_ids.shape
    v, d = embed_table.shape
    n = b * l

    fused = pl.pallas_call(
        _fuse_table_kernel,
        out_shape=jax.ShapeDtypeStruct((v, v), jnp.float32),
    )(embed_table, proj_w.T, proj_b.reshape(1, v))

    ids = input_ids.reshape(n).astype(jnp.int32)

    gather = pl.kernel(
        _gather_body,
        out_type=jax.ShapeDtypeStruct((n, v), jnp.float32),
        mesh=plsc.VectorSubcoreMesh(
            core_axis_name="c", subcore_axis_name="s",
            num_cores=_NUM_CORES, num_subcores=_NUM_SUBCORES,
        ),
        scratch_types=[
            pltpu.VMEM((_CHUNK,), jnp.int32),
            pltpu.VMEM((_CHUNK, v), jnp.float32),
            pltpu.SemaphoreType.DMA,
        ],
        compiler_params=pltpu.CompilerParams(use_tc_tiling_on_sc=False),
    )

    logits = gather(fused, ids)
    return logits.reshape(b, l, v)


# speedup vs baseline: 1.0166x; 1.0166x over previous
import functools

import jax
import jax.numpy as jnp
from jax import lax
from jax.experimental import pallas as pl
from jax.experimental.pallas import tpu as pltpu
from jax.experimental.pallas import tpu_sc as plsc

_NUM_CORES = 2
_NUM_SUBCORES = 16
_NUM_WORKERS = _NUM_CORES * _NUM_SUBCORES

_CHUNK = 40


def _fuse_table_kernel(table_ref, wt_ref, bias_ref, m_ref):
    m_ref[...] = (
        jnp.dot(table_ref[...], wt_ref[...], preferred_element_type=jnp.float32)
        + bias_ref[...]
    )


def _gather_body(m_hbm, ids_hbm, out_hbm, idx_v, rows_a, rows_b, gsa, gsb, ssa, ssb):
    wid = lax.axis_index("s") * _NUM_CORES + lax.axis_index("c")
    n = ids_hbm.shape[0]
    n_per_w = n // _NUM_WORKERS
    base = pl.multiple_of(wid * n_per_w, 8)

    pltpu.sync_copy(ids_hbm.at[pl.ds(base, n_per_w)], idx_v)

    nch = n_per_w // _CHUNK
    nph = nch // 2

    def g_start(c, buf, sem):
        off = pl.multiple_of(c * _CHUNK, 8)
        pltpu.async_copy(m_hbm.at[idx_v.at[pl.ds(off, _CHUNK)]], buf, sem)

    def g_wait(buf, sem):
        pltpu.make_async_copy(m_hbm.at[pl.ds(0, _CHUNK)], buf, sem).wait()

    def s_start(c, buf, sem):
        off = pl.multiple_of(base + c * _CHUNK, 8)
        pltpu.async_copy(buf, out_hbm.at[pl.ds(off, _CHUNK)], sem)

    def s_wait(buf, sem):
        pltpu.make_async_copy(buf, out_hbm.at[pl.ds(base, _CHUNK)], sem).wait()

    g_start(0, rows_a, gsa)
    g_wait(rows_a, gsa)
    s_start(0, rows_a, ssa)
    g_start(1, rows_b, gsb)
    g_wait(rows_b, gsb)
    s_start(1, rows_b, ssb)
    s_wait(rows_a, ssa)
    g_start(2, rows_a, gsa)

    def pair_step(i, carry):
        c0 = i * 2
        g_wait(rows_a, gsa)
        s_start(c0, rows_a, ssa)
        s_wait(rows_b, ssb)
        g_start(c0 + 1, rows_b, gsb)
        g_wait(rows_b, gsb)
        s_start(c0 + 1, rows_b, ssb)
        s_wait(rows_a, ssa)
        g_start(c0 + 2, rows_a, gsa)
        return carry

    lax.fori_loop(1, nph - 1, pair_step, 0)

    c0 = (nph - 1) * 2
    g_wait(rows_a, gsa)
    s_start(c0, rows_a, ssa)
    s_wait(rows_b, ssb)
    g_start(c0 + 1, rows_b, gsb)
    g_wait(rows_b, gsb)
    s_start(c0 + 1, rows_b, ssb)
    s_wait(rows_a, ssa)
    s_wait(rows_b, ssb)


def kernel(input_ids, embed_table, proj_w, proj_b):
    b, l = input_ids.shape
    v, d = embed_table.shape
    n = b * l

    fused = pl.pallas_call(
        _fuse_table_kernel,
        out_shape=jax.ShapeDtypeStruct((v, v), jnp.float32),
    )(embed_table, proj_w.T, proj_b.reshape(1, v))

    ids = input_ids.reshape(n).astype(jnp.int32)

    gather = pl.kernel(
        _gather_body,
        out_type=jax.ShapeDtypeStruct((n, v), jnp.float32),
        mesh=plsc.VectorSubcoreMesh(
            core_axis_name="c", subcore_axis_name="s",
            num_cores=_NUM_CORES, num_subcores=_NUM_SUBCORES,
        ),
        scratch_types=[
            pltpu.VMEM((n // _NUM_WORKERS,), jnp.int32),
            pltpu.VMEM((_CHUNK, v), jnp.float32),
            pltpu.VMEM((_CHUNK, v), jnp.float32),
            pltpu.SemaphoreType.DMA,
            pltpu.SemaphoreType.DMA,
            pltpu.SemaphoreType.DMA,
            pltpu.SemaphoreType.DMA,
        ],
        compiler_params=pltpu.CompilerParams(use_tc_tiling_on_sc=False),
    )

    logits = gather(fused, ids)
    return logits.reshape(b, l, v)


# speedup vs baseline: 1.5596x; 1.5341x over previous
import functools

import jax
import jax.numpy as jnp
from jax import lax
from jax.experimental import pallas as pl
from jax.experimental.pallas import tpu as pltpu
from jax.experimental.pallas import tpu_sc as plsc

_NUM_CORES = 2
_NUM_SUBCORES = 16
_NUM_WORKERS = _NUM_CORES * _NUM_SUBCORES

_DPAD = 128
_CHUNK = 400

_BM = 1024


def _gather_body(table_hbm, ids_hbm, x_hbm, idx_v, rows_a, rows_b, gsa, gsb, ssa, ssb):
    wid = lax.axis_index("s") * _NUM_CORES + lax.axis_index("c")
    n = ids_hbm.shape[0]
    n_per_w = n // _NUM_WORKERS
    base = pl.multiple_of(wid * n_per_w, 8)

    pltpu.sync_copy(ids_hbm.at[pl.ds(base, n_per_w)], idx_v)

    nch = n_per_w // _CHUNK
    nph = nch // 2

    def g_start(c, buf, sem):
        off = pl.multiple_of(c * _CHUNK, 8)
        pltpu.async_copy(table_hbm.at[idx_v.at[pl.ds(off, _CHUNK)]], buf, sem)

    def g_wait(buf, sem):
        pltpu.make_async_copy(table_hbm.at[pl.ds(0, _CHUNK)], buf, sem).wait()

    def s_start(c, buf, sem):
        off = pl.multiple_of(base + c * _CHUNK, 8)
        pltpu.async_copy(buf, x_hbm.at[pl.ds(off, _CHUNK)], sem)

    def s_wait(buf, sem):
        pltpu.make_async_copy(buf, x_hbm.at[pl.ds(base, _CHUNK)], sem).wait()

    g_start(0, rows_a, gsa)
    g_wait(rows_a, gsa)
    s_start(0, rows_a, ssa)
    g_start(1, rows_b, gsb)
    g_wait(rows_b, gsb)
    s_start(1, rows_b, ssb)
    s_wait(rows_a, ssa)
    g_start(2, rows_a, gsa)

    def pair_step(i, carry):
        c0 = i * 2
        g_wait(rows_a, gsa)
        s_start(c0, rows_a, ssa)
        s_wait(rows_b, ssb)
        g_start(c0 + 1, rows_b, gsb)
        g_wait(rows_b, gsb)
        s_start(c0 + 1, rows_b, ssb)
        s_wait(rows_a, ssa)
        g_start(c0 + 2, rows_a, gsa)
        return carry

    lax.fori_loop(1, nph - 1, pair_step, 0)

    c0 = (nph - 1) * 2
    g_wait(rows_a, gsa)
    s_start(c0, rows_a, ssa)
    s_wait(rows_b, ssb)
    g_start(c0 + 1, rows_b, gsb)
    g_wait(rows_b, gsb)
    s_start(c0 + 1, rows_b, ssb)
    s_wait(rows_a, ssa)
    s_wait(rows_b, ssb)


def _proj_kernel(x_ref, wt_ref, bias_ref, out_ref):
    out_ref[...] = (
        jnp.dot(x_ref[...], wt_ref[...], preferred_element_type=jnp.float32)
        + bias_ref[...]
    )


def kernel(input_ids, embed_table, proj_w, proj_b):
    b, l = input_ids.shape
    v, d = embed_table.shape
    n = b * l

    table_pad = jnp.pad(embed_table, ((0, 0), (0, _DPAD - d)))
    wt_pad = jnp.pad(proj_w.T, ((0, _DPAD - d), (0, 0)))
    ids = input_ids.reshape(n).astype(jnp.int32)

    gather = pl.kernel(
        _gather_body,
        out_type=jax.ShapeDtypeStruct((n, _DPAD), jnp.float32),
        mesh=plsc.VectorSubcoreMesh(
            core_axis_name="c", subcore_axis_name="s",
            num_cores=_NUM_CORES, num_subcores=_NUM_SUBCORES,
        ),
        scratch_types=[
            pltpu.VMEM((n // _NUM_WORKERS,), jnp.int32),
            pltpu.VMEM((_CHUNK, _DPAD), jnp.float32),
            pltpu.VMEM((_CHUNK, _DPAD), jnp.float32),
            pltpu.SemaphoreType.DMA,
            pltpu.SemaphoreType.DMA,
            pltpu.SemaphoreType.DMA,
            pltpu.SemaphoreType.DMA,
        ],
    )
    x = gather(table_pad, ids)

    logits = pl.pallas_call(
        _proj_kernel,
        grid=(n // _BM,),
        in_specs=[
            pl.BlockSpec((_BM, _DPAD), lambda i: (i, 0)),
            pl.BlockSpec((_DPAD, v), lambda i: (0, 0)),
            pl.BlockSpec((1, v), lambda i: (0, 0)),
        ],
        out_specs=pl.BlockSpec((_BM, v), lambda i: (i, 0)),
        out_shape=jax.ShapeDtypeStruct((n, v), jnp.float32),
        compiler_params=pltpu.CompilerParams(
            dimension_semantics=("arbitrary",),
        ),
    )(x, wt_pad, proj_b.reshape(1, v))

    return logits.reshape(b, l, v)


# speedup vs baseline: 5.1484x; 3.3011x over previous
import functools

import jax
import jax.numpy as jnp
from jax import lax
from jax.experimental import pallas as pl
from jax.experimental.pallas import tpu as pltpu
from jax.experimental.pallas import tpu_sc as plsc

_NUM_CORES = 2
_NUM_SUBCORES = 16
_NUM_WORKERS = _NUM_CORES * _NUM_SUBCORES

_DPAD = 128
_CHUNK = 400

_BM = 1024


def _gather_body(table_hbm, ids_hbm, x_hbm, idx_v, rows_a, rows_b, gsa, gsb, ssa, ssb):
    wid = lax.axis_index("s") * _NUM_CORES + lax.axis_index("c")
    n = ids_hbm.shape[0]
    n_per_w = n // _NUM_WORKERS
    base = pl.multiple_of(wid * n_per_w, 8)

    pltpu.sync_copy(ids_hbm.at[pl.ds(base, n_per_w)], idx_v)

    nch = n_per_w // _CHUNK
    nph = nch // 2

    def g_start(c, buf, sem):
        off = pl.multiple_of(c * _CHUNK, 8)
        pltpu.async_copy(table_hbm.at[idx_v.at[pl.ds(off, _CHUNK)]], buf, sem)

    def g_wait(buf, sem):
        pltpu.make_async_copy(table_hbm.at[pl.ds(0, _CHUNK)], buf, sem).wait()

    def s_start(c, buf, sem):
        off = pl.multiple_of(base + c * _CHUNK, 8)
        pltpu.async_copy(buf, x_hbm.at[pl.ds(off, _CHUNK)], sem)

    def s_wait(buf, sem):
        pltpu.make_async_copy(buf, x_hbm.at[pl.ds(base, _CHUNK)], sem).wait()

    g_start(0, rows_a, gsa)
    g_wait(rows_a, gsa)
    s_start(0, rows_a, ssa)
    g_start(1, rows_b, gsb)
    g_wait(rows_b, gsb)
    s_start(1, rows_b, ssb)
    s_wait(rows_a, ssa)
    g_start(2, rows_a, gsa)

    def pair_step(i, carry):
        c0 = i * 2
        g_wait(rows_a, gsa)
        s_start(c0, rows_a, ssa)
        s_wait(rows_b, ssb)
        g_start(c0 + 1, rows_b, gsb)
        g_wait(rows_b, gsb)
        s_start(c0 + 1, rows_b, ssb)
        s_wait(rows_a, ssa)
        g_start(c0 + 2, rows_a, gsa)
        return carry

    lax.fori_loop(1, nph - 1, pair_step, 0)

    c0 = (nph - 1) * 2
    g_wait(rows_a, gsa)
    s_start(c0, rows_a, ssa)
    s_wait(rows_b, ssb)
    g_start(c0 + 1, rows_b, gsb)
    g_wait(rows_b, gsb)
    s_start(c0 + 1, rows_b, ssb)
    s_wait(rows_a, ssa)
    s_wait(rows_b, ssb)


def _proj_kernel(x_ref, w_ref, bias_ref, out_ref):
    out_ref[0] = (
        lax.dot_general(
            w_ref[...],
            x_ref[0],
            dimension_numbers=(((1,), (1,)), ((), ())),
            preferred_element_type=jnp.float32,
        )
        + bias_ref[...]
    )


def kernel(input_ids, embed_table, proj_w, proj_b):
    b, l = input_ids.shape
    v, d = embed_table.shape
    n = b * l

    table_pad = jnp.pad(embed_table, ((0, 0), (0, _DPAD - d)))
    w_pad = jnp.pad(proj_w, ((0, 0), (0, _DPAD - d)))
    ids = input_ids.T.reshape(n).astype(jnp.int32)

    gather = pl.kernel(
        _gather_body,
        out_type=jax.ShapeDtypeStruct((n, _DPAD), jnp.float32),
        mesh=plsc.VectorSubcoreMesh(
            core_axis_name="c", subcore_axis_name="s",
            num_cores=_NUM_CORES, num_subcores=_NUM_SUBCORES,
        ),
        scratch_types=[
            pltpu.VMEM((n // _NUM_WORKERS,), jnp.int32),
            pltpu.VMEM((_CHUNK, _DPAD), jnp.float32),
            pltpu.VMEM((_CHUNK, _DPAD), jnp.float32),
            pltpu.SemaphoreType.DMA,
            pltpu.SemaphoreType.DMA,
            pltpu.SemaphoreType.DMA,
            pltpu.SemaphoreType.DMA,
        ],
    )
    x = gather(table_pad, ids)

    x3 = x.reshape(l, b, _DPAD)
    out_t = pl.pallas_call(
        _proj_kernel,
        grid=(l,),
        in_specs=[
            pl.BlockSpec((1, b, _DPAD), lambda i: (i, 0, 0)),
            pl.BlockSpec((v, _DPAD), lambda i: (0, 0)),
            pl.BlockSpec((v, 1), lambda i: (0, 0)),
        ],
        out_specs=pl.BlockSpec((1, v, b), lambda i: (i, 0, 0)),
        out_shape=jax.ShapeDtypeStruct((l, v, b), jnp.float32),
        compiler_params=pltpu.CompilerParams(
            dimension_semantics=("arbitrary",),
        ),
    )(x3, w_pad, proj_b.reshape(v, 1))

    return jnp.transpose(out_t, (2, 0, 1))


# speedup vs baseline: 5.2222x; 1.0143x over previous
import functools

import jax
import jax.numpy as jnp
from jax import lax
from jax.experimental import pallas as pl
from jax.experimental.pallas import tpu as pltpu
from jax.experimental.pallas import tpu_sc as plsc

_NUM_CORES = 2
_NUM_SUBCORES = 16
_NUM_WORKERS = _NUM_CORES * _NUM_SUBCORES

_DPAD = 128
_CHUNK = 200
_SPLIT = 2


def _gather_body(table_hbm, ids_hbm, x_hbm, idx_v, rows_a, rows_b, gsa, gsb, ssa, ssb):
    wid = lax.axis_index("s") * _NUM_CORES + lax.axis_index("c")
    n = ids_hbm.shape[0]
    n_per_w = n // _NUM_WORKERS
    base = pl.multiple_of(wid * n_per_w, 8)

    pltpu.sync_copy(ids_hbm.at[pl.ds(base, n_per_w)], idx_v)

    nch = n_per_w // _CHUNK
    nph = nch // 2

    def g_start(c, buf, sem):
        off = pl.multiple_of(c * _CHUNK, 8)
        pltpu.async_copy(table_hbm.at[idx_v.at[pl.ds(off, _CHUNK)]], buf, sem)

    def g_wait(buf, sem):
        pltpu.make_async_copy(table_hbm.at[pl.ds(0, _CHUNK)], buf, sem).wait()

    def s_start(c, buf, sem):
        off = pl.multiple_of(base + c * _CHUNK, 8)
        pltpu.async_copy(buf, x_hbm.at[pl.ds(off, _CHUNK)], sem)

    def s_wait(buf, sem):
        pltpu.make_async_copy(buf, x_hbm.at[pl.ds(base, _CHUNK)], sem).wait()

    g_start(0, rows_a, gsa)
    g_wait(rows_a, gsa)
    s_start(0, rows_a, ssa)
    g_start(1, rows_b, gsb)
    g_wait(rows_b, gsb)
    s_start(1, rows_b, ssb)
    s_wait(rows_a, ssa)
    g_start(2, rows_a, gsa)

    def pair_step(i, carry):
        c0 = i * 2
        g_wait(rows_a, gsa)
        s_start(c0, rows_a, ssa)
        s_wait(rows_b, ssb)
        g_start(c0 + 1, rows_b, gsb)
        g_wait(rows_b, gsb)
        s_start(c0 + 1, rows_b, ssb)
        s_wait(rows_a, ssa)
        g_start(c0 + 2, rows_a, gsa)
        return carry

    lax.fori_loop(1, nph - 1, pair_step, 0)

    c0 = (nph - 1) * 2
    g_wait(rows_a, gsa)
    s_start(c0, rows_a, ssa)
    s_wait(rows_b, ssb)
    g_start(c0 + 1, rows_b, gsb)
    g_wait(rows_b, gsb)
    s_start(c0 + 1, rows_b, ssb)
    s_wait(rows_a, ssa)
    s_wait(rows_b, ssb)


def _proj_kernel(x_ref, w_ref, bias_ref, out_ref):
    out_ref[0] = (
        lax.dot_general(
            w_ref[...],
            x_ref[0],
            dimension_numbers=(((1,), (1,)), ((), ())),
            preferred_element_type=jnp.float32,
        )
        + bias_ref[...]
    )


def _proj_update_kernel(x_ref, w_ref, bias_ref, prev_ref, out_ref):
    del prev_ref
    _proj_kernel(x_ref, w_ref, bias_ref, out_ref)


def kernel(input_ids, embed_table, proj_w, proj_b):
    b, l = input_ids.shape
    v, d = embed_table.shape
    n = b * l

    table_pad = jnp.pad(embed_table, ((0, 0), (0, _DPAD - d)))
    w_pad = jnp.pad(proj_w, ((0, 0), (0, _DPAD - d)))
    bias = proj_b.reshape(v, 1)
    ids = input_ids.T.reshape(n).astype(jnp.int32)

    l_c = l // _SPLIT
    n_c = n // _SPLIT

    gather = pl.kernel(
        _gather_body,
        out_type=jax.ShapeDtypeStruct((n_c, _DPAD), jnp.float32),
        mesh=plsc.VectorSubcoreMesh(
            core_axis_name="c", subcore_axis_name="s",
            num_cores=_NUM_CORES, num_subcores=_NUM_SUBCORES,
        ),
        scratch_types=[
            pltpu.VMEM((n_c // _NUM_WORKERS,), jnp.int32),
            pltpu.VMEM((_CHUNK, _DPAD), jnp.float32),
            pltpu.VMEM((_CHUNK, _DPAD), jnp.float32),
            pltpu.SemaphoreType.DMA,
            pltpu.SemaphoreType.DMA,
            pltpu.SemaphoreType.DMA,
            pltpu.SemaphoreType.DMA,
        ],
    )

    xs = [
        gather(table_pad, lax.slice(ids, (c * n_c,), ((c + 1) * n_c,)))
        .reshape(l_c, b, _DPAD)
        for c in range(_SPLIT)
    ]

    common = dict(
        out_shape=jax.ShapeDtypeStruct((l, v, b), jnp.float32),
        compiler_params=pltpu.CompilerParams(
            dimension_semantics=("arbitrary",),
        ),
    )
    x_spec = pl.BlockSpec((1, b, _DPAD), lambda i: (i, 0, 0))
    w_spec = pl.BlockSpec((v, _DPAD), lambda i: (0, 0))
    b_spec = pl.BlockSpec((v, 1), lambda i: (0, 0))

    out_t = pl.pallas_call(
        _proj_kernel,
        grid=(l_c,),
        in_specs=[x_spec, w_spec, b_spec],
        out_specs=pl.BlockSpec((1, v, b), lambda i: (i, 0, 0)),
        **common,
    )(xs[0], w_pad, bias)

    for c in range(1, _SPLIT):
        out_t = pl.pallas_call(
            _proj_update_kernel,
            grid=(l_c,),
            in_specs=[
                x_spec,
                w_spec,
                b_spec,
                pl.BlockSpec(memory_space=pl.ANY),
            ],
            out_specs=pl.BlockSpec(
                (1, v, b), functools.partial(lambda c, i: (c * l_c + i, 0, 0), c)
            ),
            input_output_aliases={3: 0},
            **common,
        )(xs[c], w_pad, bias, out_t)

    return jnp.transpose(out_t, (2, 0, 1))
